# baseline (speedup 1.0000x reference)
import jax
import jax.numpy as jnp
from jax.experimental import pallas as pl
from jax.experimental.pallas import tpu as pltpu


def _clocs_kernel(qp_ref, kp_ref, bx1_ref, by1_ref, bx2_ref, by2_ref, ab_ref,
                  base_ref, tib_ref, out_ref, ti_ref, cnt_ref):
    i = pl.program_id(0)
    bk = out_ref.shape[0]
    n = out_ref.shape[2]

    row8 = jax.lax.broadcasted_iota(jnp.int32, (8, 1), 0)
    r3 = row8 & 3
    is0 = r3 == 0
    is13 = (r3 & 1) == 1
    row4 = jax.lax.broadcasted_iota(jnp.int32, (4, 1), 0)
    is_k_row = (row4 & 1) == 0

    ch = 512
    full = (n // ch) * ch
    offs = [(o, ch) for o in range(0, full, ch)]
    if n % ch:
        offs.append((full, n % ch))

    acc = jnp.zeros((8, ch), jnp.int32)
    c_tail = jnp.zeros((), jnp.int32)
    for o, w in offs:
        bx1 = bx1_ref[:, pl.ds(o, w)]
        by1 = by1_ref[:, pl.ds(o, w)]
        bx2 = bx2_ref[:, pl.ds(o, w)]
        by2 = by2_ref[:, pl.ds(o, w)]
        ab = ab_ref[:, pl.ds(o, w)]
        base = base_ref[:, pl.ds(o, w)]
        tib = tib_ref[:, pl.ds(o, w)]
        for p in range(bk // 2):
            qx1 = qp_ref[p, :, 0:1]
            qy1 = qp_ref[p, :, 1:2]
            qx2 = qp_ref[p, :, 2:3]
            qy2 = qp_ref[p, :, 3:4]
            aq = qp_ref[p, :, 4:5]
            s2 = qp_ref[p, :, 5:6]

            iw = jnp.minimum(bx2, qx2) - jnp.maximum(bx1, qx1)
            ih = jnp.minimum(by2, qy2) - jnp.maximum(by1, qy1)
            inter = iw * ih
            iou = inter / ((ab + aq) - inter)
            valid = jnp.minimum(iw, ih) > 0.0

            f02 = jnp.where(valid, jnp.where(is0, iou, s2), -10.0)
            out_ref[pl.ds(2 * p, 2), :, pl.ds(o, w)] = jnp.where(
                is13, base, f02).reshape(2, 4, w)

            kv = kp_ref[p, :, 0:1]
            ti_ref[pl.ds(2 * p, 2), :, pl.ds(o, w)] = jnp.where(
                is_k_row, kv, tib).reshape(2, 2, w)

            ones = jnp.where(valid, 1, 0)
            if w == ch:
                acc = acc + ones
            else:
                c_tail = c_tail + jnp.sum(ones)

    cnt_ref[0, 0, 0] = (jnp.sum(acc) + c_tail) >> 2


def _rep8(x):
    return jnp.broadcast_to(x[None, :], (8, x.shape[0]))


def kernel(boxes, query_boxes, scores_3d, scores_2d, dis_to_lidar_3d):
    n = boxes.shape[0]
    k = query_boxes.shape[0]
    bk = 8

    b = boxes
    area_b = (b[:, 2] - b[:, 0]) * (b[:, 3] - b[:, 1])
    bx1 = _rep8(b[:, 0])
    by1 = _rep8(b[:, 1])
    bx2 = _rep8(b[:, 2])
    by2 = _rep8(b[:, 3])
    ab = _rep8(area_b)
    zeros = jnp.zeros((n,), jnp.float32)
    base = jnp.concatenate([
        jnp.stack([zeros, scores_3d[:, 0], zeros, dis_to_lidar_3d[:, 0]], 0)
    ] * 2, 0)
    nio = jnp.arange(n, dtype=jnp.int32)
    tib = jnp.stack([jnp.zeros((n,), jnp.int32), nio,
                     jnp.zeros((n,), jnp.int32), nio], 0)

    area_q = (query_boxes[:, 2] - query_boxes[:, 0]) * (
        query_boxes[:, 3] - query_boxes[:, 1])
    qcols = jnp.concatenate(
        [query_boxes, area_q[:, None], scores_2d, jnp.zeros((k, 2), jnp.float32)],
        axis=1)
    qpair = jnp.repeat(qcols, 4, axis=0).reshape(k // 2, 8, 8)
    kcol = jnp.repeat(jnp.arange(k, dtype=jnp.int32), 2).reshape(k // 2, 4, 1)

    grid = k // bk
    cvec = lambda nrows: pl.BlockSpec((nrows, n), lambda i: (0, 0))
    out, ti, cnt = pl.pallas_call(
        _clocs_kernel,
        grid=(grid,),
        in_specs=[
            pl.BlockSpec((bk // 2, 8, 8), lambda i: (i, 0, 0)),
            pl.BlockSpec((bk // 2, 4, 1), lambda i: (i, 0, 0)),
            cvec(8), cvec(8), cvec(8), cvec(8), cvec(8), cvec(8),
            pl.BlockSpec((4, n), lambda i: (0, 0)),
        ],
        out_specs=[
            pl.BlockSpec((bk, 4, n), lambda i: (i, 0, 0)),
            pl.BlockSpec((bk, 2, n), lambda i: (i, 0, 0)),
            pl.BlockSpec(memory_space=pltpu.SMEM, block_shape=(1, 1, 1),
                         index_map=lambda i: (i, 0, 0)),
        ],
        out_shape=[
            jax.ShapeDtypeStruct((k, 4, n), jnp.float32),
            jax.ShapeDtypeStruct((k, 2, n), jnp.int32),
            jax.ShapeDtypeStruct((grid, 1, 1), jnp.int32),
        ],
        compiler_params=pltpu.CompilerParams(
            dimension_semantics=("parallel",)),
    )(qpair, kcol, bx1, by1, bx2, by2, ab, base, tib)

    overlaps = jnp.transpose(out, (0, 2, 1))
    tensor_index = jnp.transpose(ti, (0, 2, 1))
    return overlaps, tensor_index, jnp.sum(cnt[:, 0, 0])


# speedup vs baseline: 1.0123x; 1.0123x over previous
import jax
import jax.numpy as jnp
from jax.experimental import pallas as pl
from jax.experimental.pallas import tpu as pltpu


def _clocs_kernel(qp_ref, kp_ref, bx1_ref, by1_ref, bx2_ref, by2_ref, ab_ref,
                  base_ref, tib_ref, out_ref, ti_ref, cnt_ref):
    i = pl.program_id(0)
    bk = out_ref.shape[0]
    n = out_ref.shape[2]

    row8 = jax.lax.broadcasted_iota(jnp.int32, (8, 1), 0)
    r3 = row8 & 3
    is0 = r3 == 0
    is13 = (r3 & 1) == 1
    row4 = jax.lax.broadcasted_iota(jnp.int32, (4, 1), 0)
    is_k_row = (row4 & 1) == 0

    ch = 512
    full = (n // ch) * ch
    offs = [(o, ch) for o in range(0, full, ch)]
    if n % ch:
        offs.append((full, n % ch))

    acc = jnp.zeros((8, ch), jnp.int32)
    c_tail = jnp.zeros((), jnp.int32)
    for o, w in offs:
        bx1 = bx1_ref[:, pl.ds(o, w)]
        by1 = by1_ref[:, pl.ds(o, w)]
        bx2 = bx2_ref[:, pl.ds(o, w)]
        by2 = by2_ref[:, pl.ds(o, w)]
        ab = ab_ref[:, pl.ds(o, w)]
        base = base_ref[:, pl.ds(o, w)]
        tib = tib_ref[:, pl.ds(o, w)]
        for p in range(bk // 2):
            qx1 = qp_ref[p, :, 0:1]
            qy1 = qp_ref[p, :, 1:2]
            qx2 = qp_ref[p, :, 2:3]
            qy2 = qp_ref[p, :, 3:4]
            aq = qp_ref[p, :, 4:5]
            s2 = qp_ref[p, :, 5:6]

            iw = jnp.minimum(bx2, qx2) - jnp.maximum(bx1, qx1)
            ih = jnp.minimum(by2, qy2) - jnp.maximum(by1, qy1)
            inter = iw * ih
            iou = inter / ((ab + aq) - inter)
            valid = jnp.minimum(iw, ih) > 0.0

            f02 = jnp.where(valid, jnp.where(is0, iou, s2), -10.0)
            out_ref[pl.ds(2 * p, 2), :, pl.ds(o, w)] = jnp.where(
                is13, base, f02).reshape(2, 4, w)

            kv = kp_ref[p, :, 0:1]
            ti_ref[pl.ds(2 * p, 2), :, pl.ds(o, w)] = jnp.where(
                is_k_row, kv, tib).reshape(2, 2, w)

            ones = jnp.where(valid, 1, 0)
            if w == ch:
                acc = acc + ones
            else:
                c_tail = c_tail + jnp.sum(ones)

    c = (jnp.sum(acc) + c_tail) >> 2

    @pl.when(i == 0)
    def _init():
        cnt_ref[0, 0] = 0

    cnt_ref[0, 0] += c


def _rep8(x):
    return jnp.broadcast_to(x[None, :], (8, x.shape[0]))


def kernel(boxes, query_boxes, scores_3d, scores_2d, dis_to_lidar_3d):
    n = boxes.shape[0]
    k = query_boxes.shape[0]
    bk = 8

    b = boxes
    area_b = (b[:, 2] - b[:, 0]) * (b[:, 3] - b[:, 1])
    bx1 = _rep8(b[:, 0])
    by1 = _rep8(b[:, 1])
    bx2 = _rep8(b[:, 2])
    by2 = _rep8(b[:, 3])
    ab = _rep8(area_b)
    zeros = jnp.zeros((n,), jnp.float32)
    base = jnp.concatenate([
        jnp.stack([zeros, scores_3d[:, 0], zeros, dis_to_lidar_3d[:, 0]], 0)
    ] * 2, 0)
    nio = jnp.arange(n, dtype=jnp.int32)
    tib = jnp.stack([jnp.zeros((n,), jnp.int32), nio,
                     jnp.zeros((n,), jnp.int32), nio], 0)

    area_q = (query_boxes[:, 2] - query_boxes[:, 0]) * (
        query_boxes[:, 3] - query_boxes[:, 1])
    qcols = jnp.concatenate(
        [query_boxes, area_q[:, None], scores_2d, jnp.zeros((k, 2), jnp.float32)],
        axis=1)
    qpair = jnp.repeat(qcols, 4, axis=0).reshape(k // 2, 8, 8)
    kcol = jnp.repeat(jnp.arange(k, dtype=jnp.int32), 2).reshape(k // 2, 4, 1)

    grid = k // bk
    cvec = lambda nrows: pl.BlockSpec((nrows, n), lambda i: (0, 0))
    out, ti, cnt = pl.pallas_call(
        _clocs_kernel,
        grid=(grid,),
        in_specs=[
            pl.BlockSpec((bk // 2, 8, 8), lambda i: (i, 0, 0)),
            pl.BlockSpec((bk // 2, 4, 1), lambda i: (i, 0, 0)),
            cvec(8), cvec(8), cvec(8), cvec(8), cvec(8), cvec(8),
            pl.BlockSpec((4, n), lambda i: (0, 0)),
        ],
        out_specs=[
            pl.BlockSpec((bk, 4, n), lambda i: (i, 0, 0)),
            pl.BlockSpec((bk, 2, n), lambda i: (i, 0, 0)),
            pl.BlockSpec(memory_space=pltpu.SMEM, block_shape=(1, 1),
                         index_map=lambda i: (0, 0)),
        ],
        out_shape=[
            jax.ShapeDtypeStruct((k, 4, n), jnp.float32),
            jax.ShapeDtypeStruct((k, 2, n), jnp.int32),
            jax.ShapeDtypeStruct((1, 1), jnp.int32),
        ],
    )(qpair, kcol, bx1, by1, bx2, by2, ab, base, tib)

    overlaps = jnp.transpose(out, (0, 2, 1))
    tensor_index = jnp.transpose(ti, (0, 2, 1))
    return overlaps, tensor_index, cnt[0, 0]


# speedup vs baseline: 1.1071x; 1.0937x over previous
import jax
import jax.numpy as jnp
from jax.experimental import pallas as pl
from jax.experimental.pallas import tpu as pltpu


def _clocs_kernel(qp_ref, kp_ref, bx1_ref, by1_ref, bx2_ref, by2_ref, ab_ref,
                  base_ref, tib_ref, out_ref, ti_ref, cnt_ref, acc_ref):
    i = pl.program_id(0)
    bk = out_ref.shape[0]
    n = out_ref.shape[2]

    row8 = jax.lax.broadcasted_iota(jnp.int32, (8, 1), 0)
    r3 = row8 & 3
    is0 = r3 == 0
    is13 = (r3 & 1) == 1
    is_k_row = (row8 & 1) == 0

    ch = min(512, n)
    offs = list(range(0, n - ch + 1, ch))
    tail_dup = 0
    if n % ch:
        offs.append(n - ch)
        tail_dup = ch - n % ch
    lane = jax.lax.broadcasted_iota(jnp.int32, (8, ch), 1)

    acc = jnp.zeros((8, ch), jnp.int32)
    for o in offs:
        fresh = (lane >= tail_dup) if (tail_dup and o == n - ch) else None
        bx1 = bx1_ref[:, pl.ds(o, ch)]
        by1 = by1_ref[:, pl.ds(o, ch)]
        bx2 = bx2_ref[:, pl.ds(o, ch)]
        by2 = by2_ref[:, pl.ds(o, ch)]
        ab = ab_ref[:, pl.ds(o, ch)]
        base = base_ref[:, pl.ds(o, ch)]
        tib = tib_ref[:, pl.ds(o, ch)]
        for p in range(bk // 2):
            qx1 = qp_ref[p, :, 0:1]
            qy1 = qp_ref[p, :, 1:2]
            qx2 = qp_ref[p, :, 2:3]
            qy2 = qp_ref[p, :, 3:4]
            aq = qp_ref[p, :, 4:5]
            s2 = qp_ref[p, :, 5:6]

            iw = jnp.minimum(bx2, qx2) - jnp.maximum(bx1, qx1)
            ih = jnp.minimum(by2, qy2) - jnp.maximum(by1, qy1)
            inter = iw * ih
            iou = inter / ((ab + aq) - inter)
            valid = jnp.minimum(iw, ih) > 0.0

            f02 = jnp.where(valid, jnp.where(is0, iou, s2), -10.0)
            out_ref[pl.ds(2 * p, 2), :, pl.ds(o, ch)] = jnp.where(
                is13, base, f02).reshape(2, 4, ch)

            ones = jnp.where(valid, 1, 0)
            if fresh is not None:
                ones = jnp.where(fresh, ones, 0)
            acc = acc + ones

        for h in range(bk // 4):
            kv = kp_ref[0, h, :, 0:1]
            ti_ref[pl.ds(4 * h, 4), :, pl.ds(o, ch)] = jnp.where(
                is_k_row, kv, tib).reshape(4, 2, ch)

    @pl.when(i == 0)
    def _init():
        acc_ref[...] = acc

    @pl.when(i > 0)
    def _accum():
        acc_ref[...] += acc

    @pl.when(i == pl.num_programs(0) - 1)
    def _final():
        cnt_ref[0, 0] = jnp.sum(acc_ref[...]) >> 2


def _rep8(x):
    return jnp.broadcast_to(x[None, :], (8, x.shape[0]))


def kernel(boxes, query_boxes, scores_3d, scores_2d, dis_to_lidar_3d):
    n = boxes.shape[0]
    k = query_boxes.shape[0]
    bk = 8

    b = boxes
    area_b = (b[:, 2] - b[:, 0]) * (b[:, 3] - b[:, 1])
    bx1 = _rep8(b[:, 0])
    by1 = _rep8(b[:, 1])
    bx2 = _rep8(b[:, 2])
    by2 = _rep8(b[:, 3])
    ab = _rep8(area_b)
    zeros = jnp.zeros((n,), jnp.float32)
    base = jnp.concatenate([
        jnp.stack([zeros, scores_3d[:, 0], zeros, dis_to_lidar_3d[:, 0]], 0)
    ] * 2, 0)
    nio = jnp.arange(n, dtype=jnp.int32)
    zi = jnp.zeros((n,), jnp.int32)
    tib = jnp.stack([zi, nio, zi, nio, zi, nio, zi, nio], 0)

    area_q = (query_boxes[:, 2] - query_boxes[:, 0]) * (
        query_boxes[:, 3] - query_boxes[:, 1])
    qcols = jnp.concatenate(
        [query_boxes, area_q[:, None], scores_2d, jnp.zeros((k, 2), jnp.float32)],
        axis=1)
    qpair = jnp.repeat(qcols, 4, axis=0).reshape(k // 2, 8, 8)
    kcol = jnp.repeat(jnp.arange(k, dtype=jnp.int32), 2).reshape(
        k // bk, bk // 4, 8, 1)

    grid = k // bk
    cvec = lambda nrows: pl.BlockSpec((nrows, n), lambda i: (0, 0))
    out, ti, cnt = pl.pallas_call(
        _clocs_kernel,
        grid=(grid,),
        in_specs=[
            pl.BlockSpec((bk // 2, 8, 8), lambda i: (i, 0, 0)),
            pl.BlockSpec((1, bk // 4, 8, 1), lambda i: (i, 0, 0, 0)),
            cvec(8), cvec(8), cvec(8), cvec(8), cvec(8), cvec(8),
            pl.BlockSpec((8, n), lambda i: (0, 0)),
        ],
        out_specs=[
            pl.BlockSpec((bk, 4, n), lambda i: (i, 0, 0)),
            pl.BlockSpec((bk, 2, n), lambda i: (i, 0, 0)),
            pl.BlockSpec(memory_space=pltpu.SMEM, block_shape=(1, 1),
                         index_map=lambda i: (0, 0)),
        ],
        out_shape=[
            jax.ShapeDtypeStruct((k, 4, n), jnp.float32),
            jax.ShapeDtypeStruct((k, 2, n), jnp.int32),
            jax.ShapeDtypeStruct((1, 1), jnp.int32),
        ],
        scratch_shapes=[pltpu.VMEM((8, min(512, n)), jnp.int32)],
    )(qpair, kcol, bx1, by1, bx2, by2, ab, base, tib)

    overlaps = jnp.transpose(out, (0, 2, 1))
    tensor_index = jnp.transpose(ti, (0, 2, 1))
    return overlaps, tensor_index, cnt[0, 0]
